# Initial kernel scaffold; baseline (speedup 1.0000x reference)
import jax
import jax.numpy as jnp
from jax.experimental import pallas as pl


def kernel(x, edge_index, W_l0, b_l0, W_r0, W_l1, b_l1, W_r1):
    raise NotImplementedError("write your pallas kernel here")


# speedup vs baseline: 3.5996x; 3.5996x over previous
import functools

import jax
import jax.numpy as jnp
from jax import lax
from jax.experimental import pallas as pl
from jax.experimental.pallas import tpu as pltpu
from jax.experimental.pallas import tpu_sc as plsc

N = 10000
E = 320000
D = 128

NC = 2
NS = 16
NW = NC * NS
B = 128
CH = 80
EW = CH * B
E_PAD = NW * EW
ACC_ROWS = NS * 640


def _agg_body(with_cnt, *refs):
    if with_cnt:
        (x_hbm, src_hbm, dst_hbm, out_hbm, cnt_hbm,
         src_v, dst_v, rows_v, ones_v, zc_v, acc_sh, cnt_sh, sem) = refs
    else:
        (x_hbm, src_hbm, dst_hbm, out_hbm,
         src_v, dst_v, rows_v, ones_v, zc_v, acc_sh, cnt_sh, sem) = refs
        cnt_hbm = None
    cid = lax.axis_index("c")
    sid = lax.axis_index("s")
    wid = sid * NC + cid

    z16 = jnp.zeros((16,), jnp.float32)

    def _zero_rows(r, _):
        for c in range(D // 16):
            rows_v[r, pl.ds(c * 16, 16)] = z16
        return 0
    lax.fori_loop(0, B, _zero_rows, 0)

    def _zero_zc(r, _):
        zc_v[pl.ds(r * 16, 16)] = z16
        return 0
    lax.fori_loop(0, 640 // 16, _zero_zc, 0)
    for c in range(B // 16):
        ones_v[pl.ds(c * 16, 16)] = jnp.ones((16,), jnp.float32)

    for k in range(5):
        pltpu.sync_copy(rows_v, acc_sh.at[pl.ds(sid * 640 + k * B, B)])
    pltpu.sync_copy(zc_v, cnt_sh.at[pl.ds(sid * 640, 640)])

    pltpu.sync_copy(src_hbm.at[wid], src_v)
    pltpu.sync_copy(dst_hbm.at[wid], dst_v)

    plsc.subcore_barrier()

    def _chunk(j, _):
        pltpu.async_copy(x_hbm.at[src_v.at[j]], rows_v, sem).wait()
        pltpu.sync_copy(rows_v, acc_sh.at[dst_v.at[j]], add=True)
        if with_cnt:
            pltpu.sync_copy(ones_v, cnt_sh.at[dst_v.at[j]], add=True)
        return 0
    lax.fori_loop(0, CH, _chunk, 0)

    plsc.subcore_barrier()

    pltpu.sync_copy(acc_sh.at[pl.ds(sid * 640, 640)],
                    out_hbm.at[cid, pl.ds(sid * 640, 640)])
    if with_cnt:
        pltpu.sync_copy(cnt_sh.at[pl.ds(sid * 640, 640)],
                        cnt_hbm.at[cid, pl.ds(sid * 640, 640)])


def _make_agg(with_cnt):
    out_type = [jax.ShapeDtypeStruct((NC, ACC_ROWS, D), jnp.float32)]
    if with_cnt:
        out_type.append(jax.ShapeDtypeStruct((NC, ACC_ROWS), jnp.float32))
    return pl.kernel(
        functools.partial(_agg_body, with_cnt),
        out_type=tuple(out_type) if with_cnt else out_type[0],
        mesh=plsc.VectorSubcoreMesh(core_axis_name="c", subcore_axis_name="s"),
        scratch_types=[
            pltpu.VMEM((CH, B), jnp.int32),
            pltpu.VMEM((CH, B), jnp.int32),
            pltpu.VMEM((B, D), jnp.float32),
            pltpu.VMEM((B,), jnp.float32),
            pltpu.VMEM((640,), jnp.float32),
            pltpu.VMEM_SHARED((ACC_ROWS, D), jnp.float32),
            pltpu.VMEM_SHARED((ACC_ROWS,), jnp.float32),
            pltpu.SemaphoreType.DMA,
        ],
    )


_agg_with_cnt = _make_agg(True)
_agg_no_cnt = _make_agg(False)


def _combine_body(relu, agg_ref, cnt_ref, x_ref, wl_ref, b_ref, wr_ref, o_ref):
    acc = agg_ref[0] + agg_ref[1]
    c = cnt_ref[0] + cnt_ref[1]
    inv = 1.0 / jnp.maximum(c, 1.0)
    mean = acc * inv[:, None]
    h = lax.dot_general(mean, wl_ref[...], (((1,), (1,)), ((), ())),
                        preferred_element_type=jnp.float32)
    h = h + b_ref[...]
    h = h + lax.dot_general(x_ref[...], wr_ref[...], (((1,), (1,)), ((), ())),
                            preferred_element_type=jnp.float32)
    o_ref[...] = jnp.maximum(h, 0.0) if relu else h


_RB = 2048


def _combine(agg, cnt, x, W_l, b_l, W_r, relu):
    grid = (ACC_ROWS // _RB,)
    return pl.pallas_call(
        functools.partial(_combine_body, relu),
        grid=grid,
        in_specs=[
            pl.BlockSpec((NC, _RB, D), lambda i: (0, i, 0)),
            pl.BlockSpec((NC, _RB), lambda i: (0, i)),
            pl.BlockSpec((_RB, D), lambda i: (i, 0)),
            pl.BlockSpec((D, D), lambda i: (0, 0)),
            pl.BlockSpec((1, D), lambda i: (0, 0)),
            pl.BlockSpec((D, D), lambda i: (0, 0)),
        ],
        out_specs=pl.BlockSpec((_RB, D), lambda i: (i, 0)),
        out_shape=jax.ShapeDtypeStruct((N, D), jnp.float32),
    )(agg, cnt, x, W_l, b_l.reshape(1, D), W_r)


def kernel(x, edge_index, W_l0, b_l0, W_r0, W_l1, b_l1, W_r1):
    src = edge_index[0]
    dst = edge_index[1]
    pad = E_PAD - E
    src_p = jnp.concatenate(
        [src, jnp.zeros((pad,), jnp.int32)]).reshape(NW, CH, B)
    dst_p = jnp.concatenate(
        [dst, jnp.full((pad,), N, jnp.int32)]).reshape(NW, CH, B)

    agg0, cnt = _agg_with_cnt(x, src_p, dst_p)
    a1 = _combine(agg0, cnt, x, W_l0, b_l0, W_r0, relu=True)
    agg1 = _agg_no_cnt(a1, src_p, dst_p)
    return _combine(agg1, cnt, a1, W_l1, b_l1, W_r1, relu=False)
